# baseline (speedup 1.0000x reference)
import functools

import jax
import jax.numpy as jnp
import numpy as np
from jax import lax
from jax.experimental import pallas as pl
from jax.experimental.pallas import tpu as pltpu
from jax.experimental.pallas import tpu_sc as plsc

NUM_RELATIONS = 1000
EMB_DIM = 128
HALF = EMB_DIM // 2
B = 16384

_NC = 2
_NS = 16
_NW = _NC * _NS
_BPW = B // _NW
_C = 64
_NCHUNK = _BPW // _C


def _trig_kernel(rel_emb_ref, out_ref):
    ph = rel_emb_ref[...] * np.float32(1.0 / (2.0 * np.pi))
    out_ref[:, :HALF] = jnp.cos(ph)
    out_ref[:, HALF:] = jnp.sin(ph)


def _make_trig_table(relation_emb):
    return pl.pallas_call(
        _trig_kernel,
        out_shape=jax.ShapeDtypeStruct((NUM_RELATIONS, EMB_DIM), jnp.float32),
    )(relation_emb)


def _vsqrt(s):
    i = plsc.bitcast(s, jnp.int32)
    r = plsc.bitcast(jnp.int32(0x5F3759DF) - lax.shift_right_logical(i, 1),
                     jnp.float32)
    half_s = s * np.float32(0.5)
    for _ in range(3):
        r = r * (np.float32(1.5) - half_s * r * r)
    return s * r


def _row_sq_dist(hb, tb, trig_v, r, rl):
    trow = lax.shift_right_logical(rl, 1)
    toff = lax.shift_left(rl & jnp.int32(1), 6)
    acc = None
    for j in range(HALF // 16):
        lo = pl.ds(j * 16, 16)
        hi = pl.ds(HALF + j * 16, 16)
        re_h = hb[r, lo]
        im_h = hb[r, hi]
        re_t = tb[r, lo]
        im_t = tb[r, hi]
        pair = plsc.bitcast(trig_v[trow, pl.ds(toff + j * 16, 16)],
                            jnp.bfloat16)
        re_r, im_r = plsc.unpack(pair, format=plsc.PackFormat.INTERLEAVED)
        re_d = re_h * re_r - im_h * im_r - re_t
        im_d = re_h * im_r + im_h * re_r - im_t
        sq = re_d * re_d + im_d * im_d
        acc = sq if acc is None else acc + sq
    return acc


def _sc_score(head, rel, tail, entity_emb, trig_packed, gamma16):
    mesh = plsc.VectorSubcoreMesh(core_axis_name="c", subcore_axis_name="s")

    @functools.partial(
        pl.kernel,
        out_type=jax.ShapeDtypeStruct((B,), jnp.float32),
        mesh=mesh,
        compiler_params=pltpu.CompilerParams(needs_layout_passes=False),
        scratch_types=[
            pltpu.VMEM((NUM_RELATIONS // 2, EMB_DIM), jnp.int32),
            pltpu.VMEM((_BPW,), jnp.int32),
            pltpu.VMEM((_BPW,), jnp.int32),
            pltpu.VMEM((_BPW + 16,), jnp.int32),
            pltpu.VMEM((_C, EMB_DIM), jnp.float32),
            pltpu.VMEM((_C, EMB_DIM), jnp.float32),
            pltpu.VMEM((_C, EMB_DIM), jnp.float32),
            pltpu.VMEM((_C, EMB_DIM), jnp.float32),
            pltpu.VMEM((16,), jnp.float32),
            pltpu.VMEM((_C,), jnp.float32),
            pltpu.VMEM((_C,), jnp.float32),
            pltpu.SemaphoreType.DMA,
            pltpu.SemaphoreType.DMA,
            pltpu.SemaphoreType.DMA,
        ],
    )
    def k(ent_hbm, trig_hbm, head_hbm, rel_hbm, tail_hbm, gamma_hbm, out_hbm,
          trig_v, ihs, its, irs, hb0, tb0, hb1, tb1, gv, sv0, sv1,
          sem0, sem1, sem2):
        cid = lax.axis_index("c")
        sid = lax.axis_index("s")
        wid = sid * _NC + cid
        base = wid * _BPW

        pro = [
            pltpu.async_copy(head_hbm.at[pl.ds(base, _BPW)], ihs, sem0),
            pltpu.async_copy(tail_hbm.at[pl.ds(base, _BPW)], its, sem0),
            pltpu.async_copy(rel_hbm.at[pl.ds(base, _BPW)],
                             irs.at[pl.ds(0, _BPW)], sem0),
            pltpu.async_copy(gamma_hbm, gv, sem0),
            pltpu.async_copy(trig_hbm, trig_v, sem0),
        ]
        for cp in pro:
            cp.wait()
        g = gv[...]

        bufs = [(hb0, tb0), (hb1, tb1)]
        sems = [sem0, sem1]
        lane = lax.iota(jnp.int32, 16)

        def issue(c, bufset, sem):
            hb, tb = bufset
            s = pl.ds(c * _C, _C)
            return [
                pltpu.async_copy(ent_hbm.at[ihs.at[s]], hb, sem),
                pltpu.async_copy(ent_hbm.at[its.at[s]], tb, sem),
            ]

        svs = [sv0, sv1]
        out_cps = [None, None]
        cps = issue(0, bufs[0], sems[0])
        for c in range(_NCHUNK):
            nxt = issue(c + 1, bufs[(c + 1) % 2], sems[(c + 1) % 2]) \
                if c + 1 < _NCHUNK else None
            for cp in cps:
                cp.wait()
            hb, tb = bufs[c % 2]
            sv = svs[c % 2]
            if out_cps[c % 2] is not None:
                out_cps[c % 2].wait()
                out_cps[c % 2] = None

            def group_body(grp, carry):

                def row_body(rr, sel):
                    rl = irs[pl.ds(c * _C + grp * 16 + rr, 16)][0]
                    acc = _row_sq_dist(hb, tb, trig_v,
                                       grp * 16 + rr, rl)
                    tot = jnp.full((16,), jnp.sum(acc), jnp.float32)
                    return jnp.where(lane == rr, tot, sel)

                sel = lax.fori_loop(0, 16, row_body,
                                    jnp.zeros((16,), jnp.float32), unroll=8)
                sv[pl.ds(grp * 16, 16)] = g - _vsqrt(sel)
                return carry

            lax.fori_loop(0, _C // 16, group_body, jnp.int32(0))
            out_cps[c % 2] = pltpu.async_copy(
                sv, out_hbm.at[pl.ds(base + c * _C, _C)], sem2)
            cps = nxt
        for cp in out_cps:
            if cp is not None:
                cp.wait()

    return k(entity_emb, trig_packed, head, rel, tail, gamma16)


def kernel(head, rel, tail, entity_emb, relation_emb, gamma):
    trig = _make_trig_table(relation_emb)
    c = trig[:, :HALF].reshape(NUM_RELATIONS, HALF // 16, 16)
    s = trig[:, HALF:].reshape(NUM_RELATIONS, HALF // 16, 16)
    pairs = jnp.stack([c, s], axis=-1).astype(jnp.bfloat16)
    trig_packed = lax.bitcast_convert_type(
        pairs, jnp.int32).reshape(NUM_RELATIONS // 2, EMB_DIM)
    gamma16 = jnp.broadcast_to(gamma, (16,))
    return _sc_score(head, rel, tail, entity_emb, trig_packed, gamma16)


# speedup vs baseline: 1.0275x; 1.0275x over previous
import functools

import jax
import jax.numpy as jnp
import numpy as np
from jax import lax
from jax.experimental import pallas as pl
from jax.experimental.pallas import tpu as pltpu
from jax.experimental.pallas import tpu_sc as plsc

NUM_RELATIONS = 1000
EMB_DIM = 128
HALF = EMB_DIM // 2
B = 16384

_NC = 2
_NS = 16
_NW = _NC * _NS
_BPW = B // _NW
_C = 128
_NCHUNK = _BPW // _C


def _trig_kernel(rel_emb_ref, out_ref):
    ph = rel_emb_ref[...] * np.float32(1.0 / (2.0 * np.pi))
    out_ref[:, :HALF] = jnp.cos(ph)
    out_ref[:, HALF:] = jnp.sin(ph)


def _make_trig_table(relation_emb):
    return pl.pallas_call(
        _trig_kernel,
        out_shape=jax.ShapeDtypeStruct((NUM_RELATIONS, EMB_DIM), jnp.float32),
    )(relation_emb)


def _vsqrt(s):
    i = plsc.bitcast(s, jnp.int32)
    r = plsc.bitcast(jnp.int32(0x5F3759DF) - lax.shift_right_logical(i, 1),
                     jnp.float32)
    half_s = s * np.float32(0.5)
    for _ in range(3):
        r = r * (np.float32(1.5) - half_s * r * r)
    return s * r


def _row_sq_dist(hb, tb, trig_v, r, rl):
    trow = lax.shift_right_logical(rl, 1)
    toff = lax.shift_left(rl & jnp.int32(1), 6)
    acc = None
    for j in range(HALF // 16):
        lo = pl.ds(j * 16, 16)
        hi = pl.ds(HALF + j * 16, 16)
        re_h = hb[r, lo]
        im_h = hb[r, hi]
        re_t = tb[r, lo]
        im_t = tb[r, hi]
        pair = plsc.bitcast(trig_v[trow, pl.ds(toff + j * 16, 16)],
                            jnp.bfloat16)
        re_r, im_r = plsc.unpack(pair, format=plsc.PackFormat.INTERLEAVED)
        re_d = re_h * re_r - im_h * im_r - re_t
        im_d = re_h * im_r + im_h * re_r - im_t
        sq = re_d * re_d + im_d * im_d
        acc = sq if acc is None else acc + sq
    return acc


def _sc_score(head, rel, tail, entity_emb, trig_packed, gamma16):
    mesh = plsc.VectorSubcoreMesh(core_axis_name="c", subcore_axis_name="s")

    @functools.partial(
        pl.kernel,
        out_type=jax.ShapeDtypeStruct((B,), jnp.float32),
        mesh=mesh,
        compiler_params=pltpu.CompilerParams(needs_layout_passes=False),
        scratch_types=[
            pltpu.VMEM((NUM_RELATIONS // 2, EMB_DIM), jnp.int32),
            pltpu.VMEM((_C,), jnp.int32),
            pltpu.VMEM((_C,), jnp.int32),
            pltpu.VMEM((_C,), jnp.int32),
            pltpu.VMEM((_C,), jnp.int32),
            pltpu.VMEM((_C,), jnp.int32),
            pltpu.VMEM((_C,), jnp.int32),
            pltpu.VMEM((_C, EMB_DIM), jnp.float32),
            pltpu.VMEM((_C, EMB_DIM), jnp.float32),
            pltpu.VMEM((_C, EMB_DIM), jnp.float32),
            pltpu.VMEM((_C, EMB_DIM), jnp.float32),
            pltpu.VMEM((16,), jnp.float32),
            pltpu.VMEM((_C,), jnp.float32),
            pltpu.SemaphoreType.DMA,
            pltpu.SemaphoreType.DMA,
            pltpu.SemaphoreType.DMA,
            pltpu.SemaphoreType.DMA,
            pltpu.SemaphoreType.DMA,
        ],
    )
    def k(ent_hbm, trig_hbm, head_hbm, rel_hbm, tail_hbm, gamma_hbm, out_hbm,
          trig_v, irc0, irc1, ihc0, ihc1, itc0, itc1, hb0, tb0, hb1, tb1,
          gv, sv, sem_pro, sem_idx, semg0, semg1, sem_out):
        cid = lax.axis_index("c")
        sid = lax.axis_index("s")
        wid = sid * _NC + cid
        base = wid * _BPW

        ihc = [ihc0, ihc1]
        itc = [itc0, itc1]
        irc = [irc0, irc1]
        hbs = [hb0, hb1]
        tbs = [tb0, tb1]
        semg = [semg0, semg1]

        def idx_copies(c):
            par = c % 2
            s = pl.ds(base + c * _C, _C)
            return [
                pltpu.async_copy(head_hbm.at[s], ihc[par], sem_idx),
                pltpu.async_copy(tail_hbm.at[s], itc[par], sem_idx),
                pltpu.async_copy(rel_hbm.at[s], irc[par], sem_idx),
            ]

        def gathers(c):
            par = c % 2
            return [
                pltpu.async_copy(ent_hbm.at[ihc[par]], hbs[par], semg[par]),
                pltpu.async_copy(ent_hbm.at[itc[par]], tbs[par], semg[par]),
            ]

        other = [
            pltpu.async_copy(gamma_hbm, gv, sem_pro),
            pltpu.async_copy(trig_hbm, trig_v, sem_pro),
        ]
        i0 = idx_copies(0)
        i1 = idx_copies(1)
        for cp in i0:
            cp.wait()
        g0 = gathers(0)
        for cp in i1:
            cp.wait()
        g1 = gathers(1)
        for cp in other:
            cp.wait()
        g = gv[...]

        lane = lax.iota(jnp.int32, 16)
        bitmasks = [(lane & jnp.int32(1 << b)) != 0 for b in range(4)]

        pending_g = [g0, g1]
        pending_i = [None, None]
        out_cp = None
        for c in range(_NCHUNK):
            par = c % 2
            for cp in pending_g[par]:
                cp.wait()
            if c + 2 < _NCHUNK:
                pending_i[par] = idx_copies(c + 2)
            if out_cp is not None:
                out_cp.wait()
                out_cp = None
            hb, tb = hbs[par], tbs[par]

            def group_body(grp, carry):
                rel16 = irc[par][pl.ds(grp * 16, 16)]
                partials = {}
                for rr in range(16):
                    acc = _row_sq_dist(hb, tb, trig_v,
                                       grp * 16 + jnp.int32(rr), rel16[rr])
                    v = jnp.full((16,), jnp.sum(acc), jnp.float32)
                    lvl = 0
                    while lvl in partials:
                        v = jnp.where(bitmasks[lvl], v, partials.pop(lvl))
                        lvl += 1
                    partials[lvl] = v
                sv[pl.ds(grp * 16, 16)] = g - _vsqrt(partials[4])
                return carry

            lax.fori_loop(0, _C // 16, group_body, jnp.int32(0))
            out_cp = pltpu.async_copy(
                sv, out_hbm.at[pl.ds(base + c * _C, _C)], sem_out)
            if c + 2 < _NCHUNK:
                for cp in pending_i[par]:
                    cp.wait()
                pending_i[par] = None
                pending_g[par] = gathers(c + 2)
        if out_cp is not None:
            out_cp.wait()

    return k(entity_emb, trig_packed, head, rel, tail, gamma16)


def kernel(head, rel, tail, entity_emb, relation_emb, gamma):
    trig = _make_trig_table(relation_emb)
    c = trig[:, :HALF].reshape(NUM_RELATIONS, HALF // 16, 16)
    s = trig[:, HALF:].reshape(NUM_RELATIONS, HALF // 16, 16)
    pairs = jnp.stack([c, s], axis=-1).astype(jnp.bfloat16)
    trig_packed = lax.bitcast_convert_type(
        pairs, jnp.int32).reshape(NUM_RELATIONS // 2, EMB_DIM)
    gamma16 = jnp.broadcast_to(gamma, (16,))
    return _sc_score(head, rel, tail, entity_emb, trig_packed, gamma16)


# speedup vs baseline: 1.0881x; 1.0590x over previous
import functools

import jax
import jax.numpy as jnp
import numpy as np
from jax import lax
from jax.experimental import pallas as pl
from jax.experimental.pallas import tpu as pltpu
from jax.experimental.pallas import tpu_sc as plsc

NUM_RELATIONS = 1000
EMB_DIM = 128
HALF = EMB_DIM // 2
B = 16384

_NC = 2
_NS = 16
_NW = _NC * _NS
_BPW = B // _NW
_C = 128
_NCHUNK = _BPW // _C


def _trig_kernel(rel_emb_ref, out_ref):
    ph = rel_emb_ref[...] * np.float32(1.0 / (2.0 * np.pi))
    out_ref[:, :HALF] = jnp.cos(ph)
    out_ref[:, HALF:] = jnp.sin(ph)


def _make_trig_table(relation_emb):
    return pl.pallas_call(
        _trig_kernel,
        out_shape=jax.ShapeDtypeStruct((NUM_RELATIONS, EMB_DIM), jnp.float32),
    )(relation_emb)


def _vsqrt(s):
    i = plsc.bitcast(s, jnp.int32)
    r = plsc.bitcast(jnp.int32(0x5F3759DF) - lax.shift_right_logical(i, 1),
                     jnp.float32)
    half_s = s * np.float32(0.5)
    for _ in range(3):
        r = r * (np.float32(1.5) - half_s * r * r)
    return s * r


def _row_sq_dist(eb, rb, r):
    rt = r + jnp.int32(_C)
    acc = None
    for j in range(HALF // 16):
        lo = pl.ds(j * 16, 16)
        hi = pl.ds(HALF + j * 16, 16)
        re_h = eb[r, lo]
        im_h = eb[r, hi]
        re_t = eb[rt, lo]
        im_t = eb[rt, hi]
        re_r = rb[r, lo]
        im_r = rb[r, hi]
        re_d = re_h * re_r - im_h * im_r - re_t
        im_d = re_h * im_r + im_h * re_r - im_t
        sq = re_d * re_d + im_d * im_d
        acc = sq if acc is None else acc + sq
    return acc


def _sc_score(head, rel, tail, entity_emb, trig, gamma16, comb_idx):
    mesh = plsc.VectorSubcoreMesh(core_axis_name="c", subcore_axis_name="s")

    @functools.partial(
        pl.kernel,
        out_type=jax.ShapeDtypeStruct((B,), jnp.float32),
        mesh=mesh,
        compiler_params=pltpu.CompilerParams(needs_layout_passes=False),
        scratch_types=[
            pltpu.VMEM((2 * _BPW,), jnp.int32),
            pltpu.VMEM((_BPW,), jnp.int32),
            pltpu.VMEM((2 * _C, EMB_DIM), jnp.float32),
            pltpu.VMEM((_C, EMB_DIM), jnp.float32),
            pltpu.VMEM((2 * _C, EMB_DIM), jnp.float32),
            pltpu.VMEM((_C, EMB_DIM), jnp.float32),
            pltpu.VMEM((16,), jnp.float32),
            pltpu.VMEM((_C,), jnp.float32),
            pltpu.VMEM((_C,), jnp.float32),
            pltpu.SemaphoreType.DMA,
            pltpu.SemaphoreType.DMA,
            pltpu.SemaphoreType.DMA,
        ],
    )
    def k(ent_hbm, trig_hbm, comb_hbm, rel_hbm, gamma_hbm, out_hbm,
          ics, irs, eb0, rb0, eb1, rb1, gv, sv0, sv1,
          sem0, sem1, sem2):
        cid = lax.axis_index("c")
        sid = lax.axis_index("s")
        wid = sid * _NC + cid
        base = wid * _BPW

        pro = [
            pltpu.async_copy(comb_hbm.at[pl.ds(2 * base, 2 * _BPW)],
                             ics, sem0),
            pltpu.async_copy(rel_hbm.at[pl.ds(base, _BPW)], irs, sem0),
            pltpu.async_copy(gamma_hbm, gv, sem0),
        ]
        for cp in pro:
            cp.wait()
        g = gv[...]

        bufs = [(eb0, rb0), (eb1, rb1)]
        sems = [sem0, sem1]

        lane = lax.iota(jnp.int32, 16)
        bitmasks = [(lane & jnp.int32(1 << b)) != 0 for b in range(4)]

        def issue(c, bufset, sem):
            eb, rb = bufset
            return [
                pltpu.async_copy(
                    ent_hbm.at[ics.at[pl.ds(c * 2 * _C, 2 * _C)]], eb, sem),
                pltpu.async_copy(
                    trig_hbm.at[irs.at[pl.ds(c * _C, _C)]], rb, sem),
            ]

        svs = [sv0, sv1]
        out_cps = [None, None]
        cps = issue(0, bufs[0], sems[0])
        for c in range(_NCHUNK):
            nxt = issue(c + 1, bufs[(c + 1) % 2], sems[(c + 1) % 2]) \
                if c + 1 < _NCHUNK else None
            for cp in cps:
                cp.wait()
            eb, rb = bufs[c % 2]
            sv = svs[c % 2]
            if out_cps[c % 2] is not None:
                out_cps[c % 2].wait()
                out_cps[c % 2] = None

            def group_body(grp, carry):
                partials = {}
                for rr in range(16):
                    acc = _row_sq_dist(eb, rb, grp * 16 + jnp.int32(rr))
                    v = jnp.full((16,), jnp.sum(acc), jnp.float32)
                    lvl = 0
                    while lvl in partials:
                        v = jnp.where(bitmasks[lvl], v, partials.pop(lvl))
                        lvl += 1
                    partials[lvl] = v
                sel = partials[4]
                sv[pl.ds(grp * 16, 16)] = g - _vsqrt(sel)
                return carry

            lax.fori_loop(0, _C // 16, group_body, jnp.int32(0))
            out_cps[c % 2] = pltpu.async_copy(
                sv, out_hbm.at[pl.ds(base + c * _C, _C)], sem2)
            cps = nxt
        for cp in out_cps:
            if cp is not None:
                cp.wait()

    return k(entity_emb, trig, comb_idx, rel, gamma16)


def kernel(head, rel, tail, entity_emb, relation_emb, gamma):
    trig = _make_trig_table(relation_emb)
    comb_idx = jnp.stack(
        [head.reshape(_NW, _NCHUNK, _C), tail.reshape(_NW, _NCHUNK, _C)],
        axis=2).reshape(-1)
    gamma16 = jnp.broadcast_to(gamma, (16,))
    return _sc_score(head, rel, tail, entity_emb, trig, gamma16, comb_idx)


# speedup vs baseline: 1.1325x; 1.0408x over previous
import functools

import jax
import jax.numpy as jnp
import numpy as np
from jax import lax
from jax.experimental import pallas as pl
from jax.experimental.pallas import tpu as pltpu
from jax.experimental.pallas import tpu_sc as plsc

NUM_RELATIONS = 1000
EMB_DIM = 128
HALF = EMB_DIM // 2
B = 16384

_NC = 2
_NS = 16
_NW = _NC * _NS
_BPW = B // _NW
_C = 128
_NCHUNK = _BPW // _C


def _trig_kernel(rel_emb_ref, out_ref):
    ph = rel_emb_ref[...] * np.float32(1.0 / (2.0 * np.pi))
    out_ref[:, :HALF] = jnp.cos(ph)
    out_ref[:, HALF:] = jnp.sin(ph)


def _make_trig_table(relation_emb):
    return pl.pallas_call(
        _trig_kernel,
        out_shape=jax.ShapeDtypeStruct((NUM_RELATIONS, EMB_DIM), jnp.float32),
    )(relation_emb)


def _vsqrt(s):
    i = plsc.bitcast(s, jnp.int32)
    r = plsc.bitcast(jnp.int32(0x5F3759DF) - lax.shift_right_logical(i, 1),
                     jnp.float32)
    half_s = s * np.float32(0.5)
    for _ in range(3):
        r = r * (np.float32(1.5) - half_s * r * r)
    return s * r


def _row_sq_dist(hb, tb, rb, r):
    acc = None
    for j in range(HALF // 16):
        lo = pl.ds(j * 16, 16)
        hi = pl.ds(HALF + j * 16, 16)
        re_h = hb[r, lo]
        im_h = hb[r, hi]
        re_t = tb[r, lo]
        im_t = tb[r, hi]
        re_r = rb[r, lo]
        im_r = rb[r, hi]
        re_d = re_h * re_r - im_h * im_r - re_t
        im_d = re_h * im_r + im_h * re_r - im_t
        sq = re_d * re_d + im_d * im_d
        acc = sq if acc is None else acc + sq
    return acc


def _sc_score(head, rel, tail, entity_emb, trig, gamma16):
    mesh = plsc.VectorSubcoreMesh(core_axis_name="c", subcore_axis_name="s")

    @functools.partial(
        pl.kernel,
        out_type=jax.ShapeDtypeStruct((B,), jnp.float32),
        mesh=mesh,
        compiler_params=pltpu.CompilerParams(needs_layout_passes=False),
        scratch_types=[
            pltpu.VMEM((_BPW,), jnp.int32),
            pltpu.VMEM((_BPW,), jnp.int32),
            pltpu.VMEM((_BPW,), jnp.int32),
            pltpu.VMEM((_C, EMB_DIM), jnp.float32),
            pltpu.VMEM((_C, EMB_DIM), jnp.float32),
            pltpu.VMEM((_C, EMB_DIM), jnp.float32),
            pltpu.VMEM((_C, EMB_DIM), jnp.float32),
            pltpu.VMEM((_C, EMB_DIM), jnp.float32),
            pltpu.VMEM((_C, EMB_DIM), jnp.float32),
            pltpu.VMEM((16,), jnp.float32),
            pltpu.VMEM((_C,), jnp.float32),
            pltpu.VMEM((_C,), jnp.float32),
            pltpu.SemaphoreType.DMA,
            pltpu.SemaphoreType.DMA,
            pltpu.SemaphoreType.DMA,
        ],
    )
    def k(ent_hbm, trig_hbm, head_hbm, rel_hbm, tail_hbm, gamma_hbm, out_hbm,
          ihs, its, irs, hb0, tb0, rb0, hb1, tb1, rb1, gv, sv0, sv1,
          sem0, sem1, sem2):
        cid = lax.axis_index("c")
        sid = lax.axis_index("s")
        wid = sid * _NC + cid
        base = wid * _BPW

        pro = [
            pltpu.async_copy(head_hbm.at[pl.ds(base, _BPW)], ihs, sem0),
            pltpu.async_copy(tail_hbm.at[pl.ds(base, _BPW)], its, sem0),
            pltpu.async_copy(rel_hbm.at[pl.ds(base, _BPW)], irs, sem0),
            pltpu.async_copy(gamma_hbm, gv, sem0),
        ]
        for cp in pro:
            cp.wait()
        g = gv[...]

        bufs = [(hb0, tb0, rb0), (hb1, tb1, rb1)]
        sems = [sem0, sem1]

        lane = lax.iota(jnp.int32, 16)
        bitmasks = [(lane & jnp.int32(1 << b)) != 0 for b in range(4)]

        def issue(c, bufset, sem):
            hb, tb, rb = bufset
            s = pl.ds(c * _C, _C)
            return [
                pltpu.async_copy(ent_hbm.at[ihs.at[s]], hb, sem),
                pltpu.async_copy(ent_hbm.at[its.at[s]], tb, sem),
                pltpu.async_copy(trig_hbm.at[irs.at[s]], rb, sem),
            ]

        svs = [sv0, sv1]
        out_cps = [None, None]
        cps = issue(0, bufs[0], sems[0])
        for c in range(_NCHUNK):
            nxt = issue(c + 1, bufs[(c + 1) % 2], sems[(c + 1) % 2]) \
                if c + 1 < _NCHUNK else None
            for cp in cps:
                cp.wait()
            hb, tb, rb = bufs[c % 2]
            sv = svs[c % 2]
            if out_cps[c % 2] is not None:
                out_cps[c % 2].wait()
                out_cps[c % 2] = None

            def group_body(grp, carry):
                partials = {}
                for rr in range(16):
                    acc = _row_sq_dist(hb, tb, rb, grp * 16 + jnp.int32(rr))
                    v = jnp.full((16,), jnp.sum(acc), jnp.float32)
                    lvl = 0
                    while lvl in partials:
                        v = jnp.where(bitmasks[lvl], v, partials.pop(lvl))
                        lvl += 1
                    partials[lvl] = v
                sel = partials[4]
                sv[pl.ds(grp * 16, 16)] = g - _vsqrt(sel)
                return carry

            lax.fori_loop(0, _C // 16, group_body, jnp.int32(0))
            out_cps[c % 2] = pltpu.async_copy(
                sv, out_hbm.at[pl.ds(base + c * _C, _C)], sem2)
            cps = nxt
        for cp in out_cps:
            if cp is not None:
                cp.wait()

    return k(entity_emb, trig, head, rel, tail, gamma16)


def kernel(head, rel, tail, entity_emb, relation_emb, gamma):
    trig = _make_trig_table(relation_emb)
    gamma16 = jnp.broadcast_to(gamma, (16,))
    return _sc_score(head, rel, tail, entity_emb, trig, gamma16)
